# baseline (speedup 1.0000x reference)
import functools

import jax
import jax.numpy as jnp
from jax import lax
from jax.experimental import pallas as pl
from jax.experimental.pallas import tpu as pltpu
from jax.experimental.pallas import tpu_sc as plsc

N = 10000
E = 160000
D = 256

NC = 2
NS = 16
NW = NC * NS
LANES = 16

DA = D + 16
NPAD = 10240

EB = 8000
ESTEPS = E // EB

K_DEC = 40
ED = E // NW
G_DEC = ED // K_DEC
IDXCAP = 5080


def _decode_gather():
    mesh = plsc.VectorSubcoreMesh(core_axis_name="c", subcore_axis_name="s")

    scratch = [
        pltpu.VMEM((IDXCAP,), jnp.int32),
        pltpu.VMEM((IDXCAP,), jnp.int32),
        pltpu.VMEM((2, K_DEC, D), jnp.float32),
        pltpu.VMEM((2, K_DEC, D), jnp.float32),
        pltpu.VMEM((2, K_DEC, D), jnp.float32),
        pltpu.SemaphoreType.DMA,
        pltpu.SemaphoreType.DMA,
        pltpu.SemaphoreType.DMA,
        pltpu.SemaphoreType.DMA,
    ]

    @functools.partial(
        pl.kernel, mesh=mesh,
        out_type=jax.ShapeDtypeStruct((E, D), jnp.float32),
        scratch_types=scratch)
    def k(x_hbm, esrc_hbm, edst_hbm, had_hbm, si, di, av, bv, ov,
          sg0, sg1, sw0, sw1):
        c = lax.axis_index("c")
        s = lax.axis_index("s")
        w = c * NS + s
        base_e = w * ED
        sems_g = (sg0, sg1)
        sems_w = (sw0, sw1)

        pltpu.sync_copy(esrc_hbm.at[pl.ds(base_e, ED)], si.at[pl.ds(0, ED)])
        pltpu.sync_copy(edst_hbm.at[pl.ds(base_e, ED)], di.at[pl.ds(0, ED)])
        zi = jnp.zeros((LANES,), jnp.int32)
        for t in range(ED, IDXCAP, LANES):
            si[pl.ds(t, LANES)] = zi
            di[pl.ds(t, LANES)] = zi

        def fire_gather(g, p):
            pltpu.async_copy(x_hbm.at[si.at[pl.ds(g * K_DEC, K_DEC)]],
                             av.at[p], sems_g[p])
            pltpu.async_copy(x_hbm.at[di.at[pl.ds(g * K_DEC, K_DEC)]],
                             bv.at[p], sems_g[p])

        def wait_gather(g, p):
            pltpu.make_async_copy(x_hbm.at[si.at[pl.ds(g * K_DEC, K_DEC)]],
                                  av.at[p], sems_g[p]).wait()
            pltpu.make_async_copy(x_hbm.at[di.at[pl.ds(g * K_DEC, K_DEC)]],
                                  bv.at[p], sems_g[p]).wait()

        def hadamard(p):
            def hrow(r, _):
                for j in range(D // LANES):
                    sl = pl.ds(j * LANES, LANES)
                    ov[p, r, sl] = av[p, r, sl] * bv[p, r, sl]
                return _

            lax.fori_loop(0, K_DEC, hrow, None, unroll=2)

        def fire_write(g, p):
            pltpu.async_copy(
                ov.at[p], had_hbm.at[pl.ds(base_e + g * K_DEC, K_DEC)],
                sems_w[p])

        def wait_write(g, p):
            pltpu.make_async_copy(
                ov.at[p], had_hbm.at[pl.ds(base_e + g * K_DEC, K_DEC)],
                sems_w[p]).wait()

        def stage(g, p, first, fire_next=True):
            if not first:
                wait_write(g - 2, p)
            if fire_next:
                fire_gather(g + 1, 1 - p)
            wait_gather(g, p)
            hadamard(p)
            fire_write(g, p)

        fire_gather(0, 0)
        stage(0, 0, True)
        stage(1, 1, True)

        def body(gp, _):
            g = 2 * gp + 2
            stage(g, 0, False)
            stage(g + 1, 1, False)
            return _

        lax.fori_loop(0, (G_DEC - 3) // 2, body, None)
        stage(G_DEC - 1, 0, False, fire_next=False)
        wait_write(G_DEC - 2, 1)
        wait_write(G_DEC - 1, 0)

    return k


_decode = _decode_gather()


NCH = 4


def _scatter_body(src_ref, dst_ref, x_ref, *o_refs):
    @pl.when(pl.program_id(0) == 0)
    def _init():
        for o_ref in o_refs:
            o_ref[...] = jnp.zeros((NPAD, D), jnp.float32)

    def body(e, _):
        for h, o_ref in enumerate(o_refs):
            sidx = src_ref[0, 0, NCH * e + h]
            didx = dst_ref[0, 0, NCH * e + h]
            m = x_ref[pl.ds(sidx, 1), :]
            a = o_ref[pl.ds(didx, 1), :]
            o_ref[pl.ds(didx, 1), :] = a + m
        return _

    lax.fori_loop(0, EB // NCH, body, None, unroll=4)


_scatter = pl.pallas_call(
    _scatter_body,
    grid=(ESTEPS,),
    in_specs=[
        pl.BlockSpec((1, 1, EB), lambda i: (i, 0, 0), memory_space=pltpu.SMEM),
        pl.BlockSpec((1, 1, EB), lambda i: (i, 0, 0), memory_space=pltpu.SMEM),
        pl.BlockSpec((N, D), lambda i: (0, 0)),
    ],
    out_specs=[pl.BlockSpec((NPAD, D), lambda i: (0, 0))] * NCH,
    out_shape=[jax.ShapeDtypeStruct((NPAD, D), jnp.float32)] * NCH,
)


def _count_body(dst_ref, o1_ref, o2_ref):
    @pl.when(pl.program_id(0) == 0)
    def _init():
        o1_ref[...] = jnp.zeros((NPAD, 128), jnp.float32)
        o2_ref[...] = jnp.zeros((NPAD, 128), jnp.float32)

    def body(e, _):
        for h, o_ref in ((0, o1_ref), (1, o2_ref)):
            didx = dst_ref[0, 0, 2 * e + h]
            a = o_ref[pl.ds(didx, 1), :]
            o_ref[pl.ds(didx, 1), :] = a + 1.0
        return _

    lax.fori_loop(0, EB // 2, body, None, unroll=8)


_count = pl.pallas_call(
    _count_body,
    grid=(ESTEPS,),
    in_specs=[
        pl.BlockSpec((1, 1, EB), lambda i: (i, 0, 0), memory_space=pltpu.SMEM),
    ],
    out_specs=[pl.BlockSpec((NPAD, 128), lambda i: (0, 0)),
               pl.BlockSpec((NPAD, 128), lambda i: (0, 0))],
    out_shape=[jax.ShapeDtypeStruct((NPAD, 128), jnp.float32),
               jax.ShapeDtypeStruct((NPAD, 128), jnp.float32)],
)


_R = 400


def _dense_body(a1, a2, a3, a4, c1, c2, x_ref, wl_ref, bl_ref, wr_ref,
                o_ref):
    agg = (a1[...] + a2[...]) + (a3[...] + a4[...])
    cnt = c1[:, 0:1] + c2[:, 0:1]
    mean = agg / jnp.maximum(cnt, 1.0)
    y = jnp.dot(mean, wl_ref[...], preferred_element_type=jnp.float32)
    y = y + bl_ref[...]
    y = y + jnp.dot(x_ref[...], wr_ref[...],
                    preferred_element_type=jnp.float32)
    o_ref[...] = jnp.maximum(y, 0.0)


_dense = pl.pallas_call(
    _dense_body,
    grid=(N // _R,),
    in_specs=[
        pl.BlockSpec((_R, D), lambda i: (i, 0)),
        pl.BlockSpec((_R, D), lambda i: (i, 0)),
        pl.BlockSpec((_R, D), lambda i: (i, 0)),
        pl.BlockSpec((_R, D), lambda i: (i, 0)),
        pl.BlockSpec((_R, 128), lambda i: (i, 0)),
        pl.BlockSpec((_R, 128), lambda i: (i, 0)),
        pl.BlockSpec((_R, D), lambda i: (i, 0)),
        pl.BlockSpec((D, D), lambda i: (0, 0)),
        pl.BlockSpec((1, D), lambda i: (0, 0)),
        pl.BlockSpec((D, D), lambda i: (0, 0)),
    ],
    out_specs=pl.BlockSpec((_R, D), lambda i: (i, 0)),
    out_shape=jax.ShapeDtypeStruct((N, D), jnp.float32),
)

_BD = 640


def _scorer_body(h_ref, w1_ref, b1_ref, w2_ref, b2_ref, o_ref):
    h = jnp.dot(h_ref[...], w1_ref[...], preferred_element_type=jnp.float32)
    h = jnp.maximum(h + b1_ref[...], 0.0)
    sc = jnp.dot(h, w2_ref[...], preferred_element_type=jnp.float32)
    o_ref[...] = sc + b2_ref[...]


_scorer = pl.pallas_call(
    _scorer_body,
    grid=(E // _BD,),
    in_specs=[
        pl.BlockSpec((_BD, D), lambda i: (i, 0)),
        pl.BlockSpec((D, D // 2), lambda i: (0, 0)),
        pl.BlockSpec((1, D // 2), lambda i: (0, 0)),
        pl.BlockSpec((D // 2, 1), lambda i: (0, 0)),
        pl.BlockSpec((1, 1), lambda i: (0, 0)),
    ],
    out_specs=pl.BlockSpec((_BD, 1), lambda i: (i, 0)),
    out_shape=jax.ShapeDtypeStruct((E, 1), jnp.float32),
)


def kernel(emb, Wl0, bl0, Wr0, Wl1, bl1, Wr1, dW1, db1, dW2, db2, edge_index, edge):
    src = edge_index[0].reshape(ESTEPS, 1, EB)
    dst = edge_index[1].reshape(ESTEPS, 1, EB)
    esrc = edge[:, 0]
    edst = edge[:, 1]

    c1, c2 = _count(dst)
    a1 = _scatter(src, dst, emb)
    x1 = _dense(*a1, c1, c2, emb, Wl0.T, bl0.reshape(1, D), Wr0.T)
    a2 = _scatter(src, dst, x1)
    x2 = _dense(*a2, c1, c2, x1, Wl1.T, bl1.reshape(1, D), Wr1.T)
    had = _decode(x2, esrc, edst)
    score = _scorer(had, dW1.T, db1.reshape(1, D // 2),
                    dW2.T, db2.reshape(1, 1))
    return score[:, 0]


# speedup vs baseline: 1.2723x; 1.2723x over previous
import functools

import jax
import jax.numpy as jnp
from jax import lax
from jax.experimental import pallas as pl
from jax.experimental.pallas import tpu as pltpu
from jax.experimental.pallas import tpu_sc as plsc

N = 10000
E = 160000
D = 256

NC = 2
NS = 16
NW = NC * NS
LANES = 16

DA = D + 16
NPAD = 10240

EB = 8000
ESTEPS = E // EB

K_DEC = 40
ED = E // NW
G_DEC = ED // K_DEC
IDXCAP = 5080


def _decode_gather():
    mesh = plsc.VectorSubcoreMesh(core_axis_name="c", subcore_axis_name="s")

    scratch = [
        pltpu.VMEM((IDXCAP,), jnp.int32),
        pltpu.VMEM((IDXCAP,), jnp.int32),
        pltpu.VMEM((2, K_DEC, D), jnp.float32),
        pltpu.VMEM((2, K_DEC, D), jnp.float32),
        pltpu.VMEM((2, K_DEC, D), jnp.float32),
        pltpu.SemaphoreType.DMA,
        pltpu.SemaphoreType.DMA,
        pltpu.SemaphoreType.DMA,
        pltpu.SemaphoreType.DMA,
    ]

    @functools.partial(
        pl.kernel, mesh=mesh,
        out_type=jax.ShapeDtypeStruct((E, D), jnp.float32),
        scratch_types=scratch)
    def k(x_hbm, esrc_hbm, edst_hbm, had_hbm, si, di, av, bv, ov,
          sg0, sg1, sw0, sw1):
        c = lax.axis_index("c")
        s = lax.axis_index("s")
        w = c * NS + s
        base_e = w * ED
        sems_g = (sg0, sg1)
        sems_w = (sw0, sw1)

        pltpu.sync_copy(esrc_hbm.at[pl.ds(base_e, ED)], si.at[pl.ds(0, ED)])
        pltpu.sync_copy(edst_hbm.at[pl.ds(base_e, ED)], di.at[pl.ds(0, ED)])
        zi = jnp.zeros((LANES,), jnp.int32)
        for t in range(ED, IDXCAP, LANES):
            si[pl.ds(t, LANES)] = zi
            di[pl.ds(t, LANES)] = zi

        def fire_gather(g, p):
            pltpu.async_copy(x_hbm.at[si.at[pl.ds(g * K_DEC, K_DEC)]],
                             av.at[p], sems_g[p])
            pltpu.async_copy(x_hbm.at[di.at[pl.ds(g * K_DEC, K_DEC)]],
                             bv.at[p], sems_g[p])

        def wait_gather(g, p):
            pltpu.make_async_copy(x_hbm.at[si.at[pl.ds(g * K_DEC, K_DEC)]],
                                  av.at[p], sems_g[p]).wait()
            pltpu.make_async_copy(x_hbm.at[di.at[pl.ds(g * K_DEC, K_DEC)]],
                                  bv.at[p], sems_g[p]).wait()

        def hadamard(p):
            def hrow(r, _):
                for j in range(D // LANES):
                    sl = pl.ds(j * LANES, LANES)
                    ov[p, r, sl] = av[p, r, sl] * bv[p, r, sl]
                return _

            lax.fori_loop(0, K_DEC, hrow, None, unroll=2)

        def fire_write(g, p):
            pltpu.async_copy(
                ov.at[p], had_hbm.at[pl.ds(base_e + g * K_DEC, K_DEC)],
                sems_w[p])

        def wait_write(g, p):
            pltpu.make_async_copy(
                ov.at[p], had_hbm.at[pl.ds(base_e + g * K_DEC, K_DEC)],
                sems_w[p]).wait()

        def stage(g, p, first, fire_next=True):
            if not first:
                wait_write(g - 2, p)
            if fire_next:
                fire_gather(g + 1, 1 - p)
            wait_gather(g, p)
            hadamard(p)
            fire_write(g, p)

        fire_gather(0, 0)
        stage(0, 0, True)
        stage(1, 1, True)

        def body(gp, _):
            g = 2 * gp + 2
            stage(g, 0, False)
            stage(g + 1, 1, False)
            return _

        lax.fori_loop(0, (G_DEC - 3) // 2, body, None)
        stage(G_DEC - 1, 0, False, fire_next=False)
        wait_write(G_DEC - 2, 1)
        wait_write(G_DEC - 1, 0)

    return k


_decode = _decode_gather()


def _scatter_body(src_ref, dst_ref, x_ref, o1_ref, o2_ref):
    @pl.when(pl.program_id(0) == 0)
    def _init():
        o1_ref[...] = jnp.zeros((NPAD, DA), jnp.float32)
        o2_ref[...] = jnp.zeros((NPAD, DA), jnp.float32)

    def body(e, _):
        for h, o_ref in ((0, o1_ref), (1, o2_ref)):
            sidx = src_ref[0, 0, 2 * e + h]
            didx = dst_ref[0, 0, 2 * e + h]
            m = x_ref[pl.ds(sidx, 1), :]
            a = o_ref[pl.ds(didx, 1), :]
            o_ref[pl.ds(didx, 1), :] = a + m
        return _

    lax.fori_loop(0, EB // 2, body, None, unroll=16)


_scatter = pl.pallas_call(
    _scatter_body,
    grid=(ESTEPS,),
    in_specs=[
        pl.BlockSpec((1, 1, EB), lambda i: (i, 0, 0), memory_space=pltpu.SMEM),
        pl.BlockSpec((1, 1, EB), lambda i: (i, 0, 0), memory_space=pltpu.SMEM),
        pl.BlockSpec((N, DA), lambda i: (0, 0)),
    ],
    out_specs=[pl.BlockSpec((NPAD, DA), lambda i: (0, 0)),
               pl.BlockSpec((NPAD, DA), lambda i: (0, 0))],
    out_shape=[jax.ShapeDtypeStruct((NPAD, DA), jnp.float32),
               jax.ShapeDtypeStruct((NPAD, DA), jnp.float32)],
)


_R = 400


def _make_dense(aug_out):
    def body(agg_ref, agg2_ref, x_ref, wl_ref, bl_ref, wr_ref, o_ref):
        agg = agg_ref[...] + agg2_ref[...]
        cnt = agg[:, D:D + 1]
        mean = agg[:, :D] / jnp.maximum(cnt, 1.0)
        y = jnp.dot(mean, wl_ref[...], preferred_element_type=jnp.float32)
        y = y + bl_ref[...]
        y = y + jnp.dot(x_ref[:, :D], wr_ref[...],
                        preferred_element_type=jnp.float32)
        y = jnp.maximum(y, 0.0)
        if aug_out:
            o_ref[:, :D] = y
            o_ref[:, D:D + 1] = jnp.ones((_R, 1), jnp.float32)
            o_ref[:, D + 1:] = jnp.zeros((_R, DA - D - 1), jnp.float32)
        else:
            o_ref[...] = y

    width = DA if aug_out else D
    return pl.pallas_call(
        body,
        grid=(N // _R,),
        in_specs=[
            pl.BlockSpec((_R, DA), lambda i: (i, 0)),
            pl.BlockSpec((_R, DA), lambda i: (i, 0)),
            pl.BlockSpec((_R, DA), lambda i: (i, 0)),
            pl.BlockSpec((D, D), lambda i: (0, 0)),
            pl.BlockSpec((1, D), lambda i: (0, 0)),
            pl.BlockSpec((D, D), lambda i: (0, 0)),
        ],
        out_specs=pl.BlockSpec((_R, width), lambda i: (i, 0)),
        out_shape=jax.ShapeDtypeStruct((N, width), jnp.float32),
    )


_dense_aug = _make_dense(True)
_dense_plain = _make_dense(False)

_BD = 640


def _scorer_body(h_ref, w1_ref, b1_ref, w2_ref, b2_ref, o_ref):
    h = jnp.dot(h_ref[...], w1_ref[...], preferred_element_type=jnp.float32)
    h = jnp.maximum(h + b1_ref[...], 0.0)
    sc = jnp.dot(h, w2_ref[...], preferred_element_type=jnp.float32)
    o_ref[...] = sc + b2_ref[...]


_scorer = pl.pallas_call(
    _scorer_body,
    grid=(E // _BD,),
    in_specs=[
        pl.BlockSpec((_BD, D), lambda i: (i, 0)),
        pl.BlockSpec((D, D // 2), lambda i: (0, 0)),
        pl.BlockSpec((1, D // 2), lambda i: (0, 0)),
        pl.BlockSpec((D // 2, 1), lambda i: (0, 0)),
        pl.BlockSpec((1, 1), lambda i: (0, 0)),
    ],
    out_specs=pl.BlockSpec((_BD, 1), lambda i: (i, 0)),
    out_shape=jax.ShapeDtypeStruct((E, 1), jnp.float32),
)


def kernel(emb, Wl0, bl0, Wr0, Wl1, bl1, Wr1, dW1, db1, dW2, db2, edge_index, edge):
    src = edge_index[0].reshape(ESTEPS, 1, EB)
    dst = edge_index[1].reshape(ESTEPS, 1, EB)
    esrc = edge[:, 0]
    edst = edge[:, 1]

    pad = jnp.concatenate(
        [jnp.ones((N, 1), jnp.float32), jnp.zeros((N, DA - D - 1), jnp.float32)],
        axis=1)
    emb_aug = jnp.concatenate([emb, pad], axis=1)

    agg1a, agg1b = _scatter(src, dst, emb_aug)
    x1 = _dense_aug(agg1a[:N], agg1b[:N], emb_aug, Wl0.T,
                    bl0.reshape(1, D), Wr0.T)
    agg2a, agg2b = _scatter(src, dst, x1)
    x2 = _dense_plain(agg2a[:N], agg2b[:N], x1, Wl1.T,
                      bl1.reshape(1, D), Wr1.T)
    had = _decode(x2, esrc, edst)
    score = _scorer(had, dW1.T, db1.reshape(1, D // 2),
                    dW2.T, db2.reshape(1, 1))
    return score[:, 0]
